# baseline (speedup 1.0000x reference)
import functools

import jax
import jax.numpy as jnp
from jax import lax
from jax.experimental import pallas as pl
from jax.experimental.pallas import tpu as pltpu
from jax.experimental.pallas import tpu_sc as plsc

_LANES = 16
_CH = 64
_NC = 2
_NS = 16
_SEGW = 32


def _tc_mlp_body(w1x_ref, w1h_ref, w2_ref, b1_ref, b2_ref, x_ref, h_ref,
                 out_ref):
    hidden = jnp.maximum(
        jnp.dot(x_ref[...], w1x_ref[...], preferred_element_type=jnp.float32)
        + jnp.dot(h_ref[...], w1h_ref[...], preferred_element_type=jnp.float32)
        + b1_ref[...], 0.0)
    out_ref[...] = jnp.dot(hidden, w2_ref[...],
                           preferred_element_type=jnp.float32) + b2_ref[...]


def _extract(vec_ref, j):
    v = vec_ref[pl.ds(j, _LANES)]
    return v[0]


def _sc_segsum_body(h_hbm, ptr_hbm, out_hbm, ptr_v, buf_v, outbuf_v):
    wid = lax.axis_index("s") * _NC + lax.axis_index("c")
    s0 = wid * _SEGW
    pltpu.sync_copy(ptr_hbm.at[pl.ds(s0, 4 * _LANES)], ptr_v)

    def seg_body(j, carry):
        start = _extract(ptr_v, j)
        end = _extract(ptr_v, j + 1)
        abase = (start // 8) * 8
        nch = (end - abase + _CH - 1) // _CH

        def chunk_body(c, accs):
            off = abase + c * _CH
            pltpu.sync_copy(h_hbm.at[pl.ds(off, _CH)], buf_v)
            rlo = jnp.maximum(start - off, 0)
            rhi = jnp.minimum(end - off, _CH)

            def row_body(r, accs):
                return tuple(accs[k] + buf_v[r, pl.ds(k * _LANES, _LANES)]
                             for k in range(8))

            return lax.fori_loop(rlo, rhi, row_body, accs)

        zeros = tuple(jnp.zeros((_LANES,), jnp.float32) for _ in range(8))
        accs = lax.fori_loop(0, nch, chunk_body, zeros)
        for k in range(8):
            outbuf_v[j, pl.ds(k * _LANES, _LANES)] = accs[k]
        return carry

    lax.fori_loop(0, _SEGW, seg_body, 0)
    pltpu.sync_copy(outbuf_v, out_hbm.at[pl.ds(s0, _SEGW)])


def kernel(h_node, x, ptr, W1, b1, W2, b2):
    n, embed_dim = h_node.shape
    nfeat = x.shape[1]
    nseg = ptr.shape[0] - 1
    hidden_dim = W1.shape[1]

    block_rows = 512
    grid = (n // block_rows,)
    w1x = W1[:nfeat]
    w1h = W1[nfeat:]

    h = pl.pallas_call(
        _tc_mlp_body,
        grid=grid,
        in_specs=[
            pl.BlockSpec((nfeat, hidden_dim), lambda i: (0, 0)),
            pl.BlockSpec((embed_dim, hidden_dim), lambda i: (0, 0)),
            pl.BlockSpec((hidden_dim, embed_dim), lambda i: (0, 0)),
            pl.BlockSpec((1, hidden_dim), lambda i: (0, 0)),
            pl.BlockSpec((1, embed_dim), lambda i: (0, 0)),
            pl.BlockSpec((block_rows, nfeat), lambda i: (i, 0)),
            pl.BlockSpec((block_rows, embed_dim), lambda i: (i, 0)),
        ],
        out_specs=pl.BlockSpec((block_rows, embed_dim), lambda i: (i, 0)),
        out_shape=jax.ShapeDtypeStruct((n + _CH, embed_dim), jnp.float32),
        compiler_params=pltpu.CompilerParams(
            dimension_semantics=("arbitrary",),
        ),
    )(w1x, w1h, W2, b1.reshape(1, -1), b2.reshape(1, -1), x, h_node)

    nw = _NC * _NS
    nseg_pad = nw * _SEGW
    ptr32 = ptr.astype(jnp.int32)
    ptr_pad = jnp.concatenate(
        [ptr32, jnp.full((nseg_pad + 4 * _LANES - (nseg + 1),), n, jnp.int32)])

    mesh = plsc.VectorSubcoreMesh(core_axis_name="c", subcore_axis_name="s",
                                  num_cores=_NC, num_subcores=_NS)
    sc_out = pl.kernel(
        _sc_segsum_body,
        out_type=jax.ShapeDtypeStruct((nseg_pad, embed_dim), jnp.float32),
        mesh=mesh,
        scratch_types=[
            pltpu.VMEM((4 * _LANES,), jnp.int32),
            pltpu.VMEM((_CH, embed_dim), jnp.float32),
            pltpu.VMEM((_SEGW, embed_dim), jnp.float32),
        ],
    )(h, ptr_pad)
    return sc_out[:nseg]
